# baseline (speedup 1.0000x reference)
import functools

import jax
import jax.numpy as jnp
from jax import lax
from jax.experimental import pallas as pl
from jax.experimental.pallas import tpu as pltpu
from jax.experimental.pallas import tpu_sc as plsc

LANES = 16


def _build_tc_transpose(V, K, TN):
    @functools.partial(
        pl.pallas_call,
        grid=(pl.cdiv(V, TN),),
        in_specs=[pl.BlockSpec((K, TN), lambda i: (0, i))],
        out_specs=pl.BlockSpec((TN, K), lambda i: (i, 0)),
        out_shape=jax.ShapeDtypeStruct((V, K), jnp.float32),
    )
    def tr(x_ref, o_ref):
        o_ref[...] = x_ref[...].T

    return tr


def _build_sc_kernel(B, V, K, b_per_w, chunk, num_cores):
    n_chunks = b_per_w // chunk
    mesh = plsc.VectorSubcoreMesh(core_axis_name="c", subcore_axis_name="s")

    @functools.partial(
        pl.kernel,
        out_type=jax.ShapeDtypeStruct((B,), jnp.float32),
        mesh=mesh,
        compiler_params=pltpu.CompilerParams(needs_layout_passes=False),
        scratch_types=[
            pltpu.VMEM((b_per_w,), jnp.int32),
            pltpu.VMEM((b_per_w,), jnp.int32),
            pltpu.VMEM((b_per_w,), jnp.int32),
            pltpu.VMEM((chunk, K), jnp.float32),
            pltpu.VMEM((chunk, K), jnp.float32),
            pltpu.VMEM((chunk, K), jnp.float32),
            pltpu.VMEM((b_per_w,), jnp.float32),
            pltpu.SemaphoreType.DMA,
            pltpu.SemaphoreType.DMA,
            pltpu.SemaphoreType.DMA,
        ],
    )
    def sc_kernel(idx0_hbm, idx1_hbm, idx2_hbm, f0_hbm, f1_hbm, f2_hbm,
                  out_hbm, idx0_v, idx1_v, idx2_v, r0_v, r1_v, r2_v, out_v,
                  sem0, sem1, sem2):
        wid = lax.axis_index("s") * num_cores + lax.axis_index("c")
        base = wid * b_per_w

        pltpu.sync_copy(idx0_hbm.at[pl.ds(base, b_per_w)], idx0_v)
        pltpu.sync_copy(idx1_hbm.at[pl.ds(base, b_per_w)], idx1_v)
        pltpu.sync_copy(idx2_hbm.at[pl.ds(base, b_per_w)], idx2_v)

        lane = lax.iota(jnp.int32, LANES)
        perms = [jnp.bitwise_xor(lane, s) for s in (8, 4, 2, 1)]

        def do_chunk(c, carry0):
            off = c * chunk

            def fire(g, carry):
                iv0 = idx0_v[pl.ds(off + g * LANES, LANES)]
                iv1 = idx1_v[pl.ds(off + g * LANES, LANES)]
                iv2 = idx2_v[pl.ds(off + g * LANES, LANES)]
                for l in range(LANES):
                    b = g * LANES + l
                    pltpu.make_async_copy(
                        f0_hbm.at[pl.ds(iv0[l], 1), :],
                        r0_v.at[pl.ds(b, 1), :], sem0).start()
                    pltpu.make_async_copy(
                        f1_hbm.at[pl.ds(iv1[l], 1), :],
                        r1_v.at[pl.ds(b, 1), :], sem1).start()
                    pltpu.make_async_copy(
                        f2_hbm.at[pl.ds(iv2[l], 1), :],
                        r2_v.at[pl.ds(b, 1), :], sem2).start()
                return carry

            lax.fori_loop(0, chunk // LANES, fire, 0)

            pltpu.make_async_copy(f0_hbm.at[pl.ds(0, chunk), :], r0_v,
                                  sem0).wait()
            pltpu.make_async_copy(f1_hbm.at[pl.ds(0, chunk), :], r1_v,
                                  sem1).wait()
            pltpu.make_async_copy(f2_hbm.at[pl.ds(0, chunk), :], r2_v,
                                  sem2).wait()

            def body(g, carry):
                vec = jnp.zeros((LANES,), jnp.float32)
                for l in range(LANES):
                    b = g * LANES + l
                    acc = (r0_v[b, pl.ds(0, LANES)]
                           * r1_v[b, pl.ds(0, LANES)]
                           * r2_v[b, pl.ds(0, LANES)])
                    for j in range(1, K // LANES):
                        acc = acc + (r0_v[b, pl.ds(j * LANES, LANES)]
                                     * r1_v[b, pl.ds(j * LANES, LANES)]
                                     * r2_v[b, pl.ds(j * LANES, LANES)])
                    for p in perms:
                        acc = acc + jnp.take_along_axis(acc, p, axis=0)
                    vec = jnp.where(lane == l, acc, vec)
                out_v[pl.ds(off + g * LANES, LANES)] = vec
                return carry

            lax.fori_loop(0, chunk // LANES, body, 0)
            return carry0

        lax.fori_loop(0, n_chunks, do_chunk, 0)

        pltpu.sync_copy(out_v, out_hbm.at[pl.ds(base, b_per_w)])

    return sc_kernel


def kernel(indices, F0, F1, F2):
    B = indices.shape[0]
    V, K = F0.shape
    info = plsc.get_sparse_core_info()
    num_workers = info.num_cores * info.num_subcores
    b_per_w = B // num_workers
    idx0 = indices[:, 0]
    idx1 = indices[:, 1]
    idx2 = indices[:, 2]
    tr = _build_tc_transpose(V, K, 512)
    F0r = tr(F0.T)
    F1r = tr(F1.T)
    F2r = tr(F2.T)
    sc = _build_sc_kernel(B, V, K, b_per_w, min(b_per_w, 256), info.num_cores)
    return sc(idx0, idx1, idx2, F0r, F1r, F2r)


# speedup vs baseline: 2.4617x; 2.4617x over previous
import functools

import jax
import jax.numpy as jnp
from jax import lax
from jax.experimental import pallas as pl
from jax.experimental.pallas import tpu as pltpu
from jax.experimental.pallas import tpu_sc as plsc

LANES = 16


def _build_gather(B, V, K, b_per_w, chunk, num_cores):
    n_chunks = b_per_w // chunk
    mesh = plsc.VectorSubcoreMesh(core_axis_name="c", subcore_axis_name="s")

    @functools.partial(
        pl.kernel,
        out_type=jax.ShapeDtypeStruct((B, K), jnp.float32),
        mesh=mesh,
        compiler_params=pltpu.CompilerParams(needs_layout_passes=False),
        scratch_types=[
            pltpu.VMEM((b_per_w,), jnp.int32),
            pltpu.VMEM((chunk, K), jnp.float32),
            pltpu.SemaphoreType.DMA,
        ],
    )
    def gather_kernel(idx_hbm, f_hbm, out_hbm, idx_v, r_v, sem):
        wid = lax.axis_index("s") * num_cores + lax.axis_index("c")
        base = wid * b_per_w

        pltpu.sync_copy(idx_hbm.at[pl.ds(base, b_per_w)], idx_v)

        def do_chunk(c, carry0):
            off = c * chunk

            def fire(g, carry):
                iv = idx_v[pl.ds(off + g * LANES, LANES)]
                for l in range(LANES):
                    b = g * LANES + l
                    pltpu.make_async_copy(
                        f_hbm.at[pl.ds(iv[l], 1), :],
                        r_v.at[pl.ds(b, 1), :], sem).start()
                return carry

            lax.fori_loop(0, chunk // LANES, fire, 0)

            pltpu.make_async_copy(f_hbm.at[pl.ds(0, chunk), :], r_v,
                                  sem).wait()
            pltpu.sync_copy(r_v, out_hbm.at[pl.ds(base + off, chunk), :])
            return carry0

        lax.fori_loop(0, n_chunks, do_chunk, 0)

    return gather_kernel


def _build_combine(B, K, b_per_w, chunk, num_cores):
    n_chunks = b_per_w // chunk
    mesh = plsc.VectorSubcoreMesh(core_axis_name="c", subcore_axis_name="s")

    @functools.partial(
        pl.kernel,
        out_type=jax.ShapeDtypeStruct((B,), jnp.float32),
        mesh=mesh,
        compiler_params=pltpu.CompilerParams(needs_layout_passes=False),
        scratch_types=[
            pltpu.VMEM((chunk, K), jnp.float32),
            pltpu.VMEM((chunk, K), jnp.float32),
            pltpu.VMEM((chunk, K), jnp.float32),
            pltpu.VMEM((b_per_w,), jnp.float32),
        ],
    )
    def combine_kernel(p0_hbm, p1_hbm, p2_hbm, out_hbm, r0_v, r1_v, r2_v,
                       out_v):
        wid = lax.axis_index("s") * num_cores + lax.axis_index("c")
        base = wid * b_per_w

        lane = lax.iota(jnp.int32, LANES)
        perms = [jnp.bitwise_xor(lane, s) for s in (8, 4, 2, 1)]

        def do_chunk(c, carry9):
            off = c * chunk
            pltpu.sync_copy(p0_hbm.at[pl.ds(base + off, chunk), :], r0_v)
            pltpu.sync_copy(p1_hbm.at[pl.ds(base + off, chunk), :], r1_v)
            pltpu.sync_copy(p2_hbm.at[pl.ds(base + off, chunk), :], r2_v)

            def body(g, carry):
                vec = jnp.zeros((LANES,), jnp.float32)
                for l in range(LANES):
                    b = g * LANES + l
                    acc = (r0_v[b, pl.ds(0, LANES)]
                           * r1_v[b, pl.ds(0, LANES)]
                           * r2_v[b, pl.ds(0, LANES)])
                    for j in range(1, K // LANES):
                        acc = acc + (r0_v[b, pl.ds(j * LANES, LANES)]
                                     * r1_v[b, pl.ds(j * LANES, LANES)]
                                     * r2_v[b, pl.ds(j * LANES, LANES)])
                    for p in perms:
                        acc = acc + jnp.take_along_axis(acc, p, axis=0)
                    vec = jnp.where(lane == l, acc, vec)
                out_v[pl.ds(off + g * LANES, LANES)] = vec
                return carry

            lax.fori_loop(0, chunk // LANES, body, 0)
            return carry9

        lax.fori_loop(0, n_chunks, do_chunk, 0)

        pltpu.sync_copy(out_v, out_hbm.at[pl.ds(base, b_per_w)])

    return combine_kernel


def kernel(indices, F0, F1, F2):
    B = indices.shape[0]
    V, K = F0.shape
    info = plsc.get_sparse_core_info()
    num_workers = info.num_cores * info.num_subcores
    b_per_w = B // num_workers
    gather = _build_gather(B, V, K, b_per_w, min(b_per_w, 256),
                           info.num_cores)
    combine = _build_combine(B, K, b_per_w, min(b_per_w, 256),
                             info.num_cores)
    p0 = gather(indices[:, 0], F0)
    p1 = gather(indices[:, 1], F1)
    p2 = gather(indices[:, 2], F2)
    return combine(p0, p1, p2)


# speedup vs baseline: 2.7461x; 1.1155x over previous
import functools

import jax
import jax.numpy as jnp
from jax import lax
from jax.experimental import pallas as pl
from jax.experimental.pallas import tpu as pltpu
from jax.experimental.pallas import tpu_sc as plsc

LANES = 16


def _build_sc_kernel(B, V, K, b_per_w, chunk, num_cores):
    n_chunks = b_per_w // chunk
    mesh = plsc.VectorSubcoreMesh(core_axis_name="c", subcore_axis_name="s")

    @functools.partial(
        pl.kernel,
        out_type=jax.ShapeDtypeStruct((B,), jnp.float32),
        mesh=mesh,
        compiler_params=pltpu.CompilerParams(needs_layout_passes=False),
        scratch_types=[
            pltpu.VMEM((b_per_w,), jnp.int32),
            pltpu.VMEM((b_per_w,), jnp.int32),
            pltpu.VMEM((b_per_w,), jnp.int32),
            pltpu.VMEM((chunk, K), jnp.float32),
            pltpu.VMEM((chunk, K), jnp.float32),
            pltpu.VMEM((chunk, K), jnp.float32),
            pltpu.VMEM((b_per_w,), jnp.float32),
            pltpu.SemaphoreType.DMA,
            pltpu.SemaphoreType.DMA,
            pltpu.SemaphoreType.DMA,
        ],
    )
    def sc_kernel(idx0_hbm, idx1_hbm, idx2_hbm, f0_hbm, f1_hbm, f2_hbm,
                  out_hbm, idx0_v, idx1_v, idx2_v, r0_v, r1_v, r2_v, out_v,
                  sem0, sem1, sem2):
        wid = lax.axis_index("s") * num_cores + lax.axis_index("c")
        base = wid * b_per_w

        pltpu.sync_copy(idx0_hbm.at[pl.ds(base, b_per_w)], idx0_v)
        pltpu.sync_copy(idx1_hbm.at[pl.ds(base, b_per_w)], idx1_v)
        pltpu.sync_copy(idx2_hbm.at[pl.ds(base, b_per_w)], idx2_v)

        lane = lax.iota(jnp.int32, LANES)
        perms = [jnp.bitwise_xor(lane, s) for s in (8, 4, 2, 1)]

        def do_chunk(c, carry0):
            off = c * chunk

            def fire(g, carry):
                iv0 = idx0_v[pl.ds(off + g * LANES, LANES)]
                iv1 = idx1_v[pl.ds(off + g * LANES, LANES)]
                iv2 = idx2_v[pl.ds(off + g * LANES, LANES)]
                for l in range(LANES):
                    b = g * LANES + l
                    pltpu.make_async_copy(
                        f0_hbm.at[pl.ds(iv0[l], 1), :],
                        r0_v.at[pl.ds(b, 1), :], sem0).start()
                    pltpu.make_async_copy(
                        f1_hbm.at[pl.ds(iv1[l], 1), :],
                        r1_v.at[pl.ds(b, 1), :], sem1).start()
                    pltpu.make_async_copy(
                        f2_hbm.at[pl.ds(iv2[l], 1), :],
                        r2_v.at[pl.ds(b, 1), :], sem2).start()
                return carry

            lax.fori_loop(0, chunk // LANES, fire, 0)

            pltpu.make_async_copy(f0_hbm.at[pl.ds(0, chunk), :], r0_v,
                                  sem0).wait()
            pltpu.make_async_copy(f1_hbm.at[pl.ds(0, chunk), :], r1_v,
                                  sem1).wait()
            pltpu.make_async_copy(f2_hbm.at[pl.ds(0, chunk), :], r2_v,
                                  sem2).wait()

            def body(g, carry):
                vec = jnp.zeros((LANES,), jnp.float32)
                for l in range(LANES):
                    b = g * LANES + l
                    acc = (r0_v[b, pl.ds(0, LANES)]
                           * r1_v[b, pl.ds(0, LANES)]
                           * r2_v[b, pl.ds(0, LANES)])
                    for j in range(1, K // LANES):
                        acc = acc + (r0_v[b, pl.ds(j * LANES, LANES)]
                                     * r1_v[b, pl.ds(j * LANES, LANES)]
                                     * r2_v[b, pl.ds(j * LANES, LANES)])
                    for p in perms:
                        acc = acc + jnp.take_along_axis(acc, p, axis=0)
                    vec = jnp.where(lane == l, acc, vec)
                out_v[pl.ds(off + g * LANES, LANES)] = vec
                return carry

            lax.fori_loop(0, chunk // LANES, body, 0)
            return carry0

        lax.fori_loop(0, n_chunks, do_chunk, 0)

        pltpu.sync_copy(out_v, out_hbm.at[pl.ds(base, b_per_w)])

    return sc_kernel


def kernel(indices, F0, F1, F2):
    B = indices.shape[0]
    V, K = F0.shape
    info = plsc.get_sparse_core_info()
    num_workers = info.num_cores * info.num_subcores
    b_per_w = B // num_workers
    idx0 = indices[:, 0]
    idx1 = indices[:, 1]
    idx2 = indices[:, 2]
    sc = _build_sc_kernel(B, V, K, b_per_w, min(b_per_w, 256),
                          info.num_cores)
    return sc(idx0, idx1, idx2, F0, F1, F2)
